# baseline (speedup 1.0000x reference)
import functools

import jax
import jax.numpy as jnp
from jax import lax
from jax.experimental import pallas as pl
from jax.experimental.pallas import tpu as pltpu
from jax.experimental.pallas import tpu_sc as plsc

N = 10000
E = 320000
B = 1024
H = 128
NC = 2
NS = 16
CH = 80

_f32 = jnp.float32


def _mesh():
    return plsc.VectorSubcoreMesh(
        core_axis_name="c", subcore_axis_name="s", num_cores=NC, num_subcores=NS
    )


_NRB = (N // NS) // 8 * 8
_NTAIL = N - _NRB * NS


def _copy_rows(s, src_ref, dst_ref):
    pltpu.sync_copy(src_ref.at[pl.ds(s * _NRB, _NRB)],
                    dst_ref.at[pl.ds(s * _NRB, _NRB)])

    @pl.when(s == NS - 1)
    def _():
        pltpu.sync_copy(src_ref.at[pl.ds(_NRB * NS, _NTAIL)],
                        dst_ref.at[pl.ds(_NRB * NS, _NTAIL)])


_copy_elems = _copy_rows


_EROWS = 2560
_RPW = _EROWS // NS
_CW = 128
_SLAB = 32


def _sc_deg_body(src1, dst1, src2, dst2, zeros_n, ones_hbm,
                 o_s1, o_d1, o_s2, o_d2, hist, ones_v, zbuf, obuf, idx_v,
                 ssem0, ssem1):
    c = lax.axis_index("c")
    s = lax.axis_index("s")
    pltpu.sync_copy(ones_hbm, ones_v)
    pltpu.sync_copy(zeros_n.at[pl.ds(0, _NRB)], zbuf)
    r0 = s * _RPW
    e0 = s * _NRB

    def phase(idx_ref, out_ref):
        pltpu.sync_copy(zbuf, hist.at[pl.ds(e0, _NRB)])

        @pl.when(s == NS - 1)
        def _():
            pltpu.sync_copy(zbuf.at[pl.ds(0, _NTAIL)],
                            hist.at[pl.ds(_NRB * NS, _NTAIL)])
        plsc.subcore_barrier()

        def start(j, sem):
            pltpu.async_copy(ones_v, hist.at[idx_v.at[j]], sem, add=True)

        def wait(j, sem):
            pltpu.make_async_copy(ones_v, hist.at[idx_v.at[j]], sem).wait()

        def seg(si, carry):
            pltpu.sync_copy(idx_ref.at[pl.ds(r0 + si * _SLAB, _SLAB)], idx_v)
            start(0, ssem0)

            def body2(t, carry2):
                j0 = 2 * t
                j1 = j0 + 1
                start(j1, ssem1)
                wait(j0, ssem0)

                @pl.when(j0 + 2 < _SLAB)
                def _():
                    start(j0 + 2, ssem0)
                wait(j1, ssem1)
                return carry2
            lax.fori_loop(0, _SLAB // 2, body2, 0)
            return carry
        lax.fori_loop(0, _RPW // _SLAB, seg, 0)
        plsc.subcore_barrier()
        pltpu.sync_copy(hist.at[pl.ds(e0, _NRB)], obuf)
        pltpu.sync_copy(obuf, out_ref.at[pl.ds(e0, _NRB)])

        @pl.when(s == NS - 1)
        def _():
            pltpu.sync_copy(hist.at[pl.ds(_NRB * NS, _NTAIL)],
                            obuf.at[pl.ds(0, _NTAIL)])
            pltpu.sync_copy(obuf.at[pl.ds(0, _NTAIL)],
                            out_ref.at[pl.ds(_NRB * NS, _NTAIL)])

    @pl.when(c == 0)
    def _():
        phase(src1, o_s1)
        phase(dst1, o_d1)

    @pl.when(c == 1)
    def _():
        phase(src2, o_s2)
        phase(dst2, o_d2)


_NSPLIT = 2


def _sc_agg_body(hs1, hs2, src1, dst1, src2, dst2, zeros_nh,
                 o1, o2, acc, rows0, rows1, sidx_v, didx_v, gsem0, gsem1):
    c = lax.axis_index("c")
    s = lax.axis_index("s")
    _copy_rows(s, zeros_nh, acc)
    r0 = s * _RPW
    rows = (rows0, rows1)
    gsems = (gsem0, gsem1)
    part = _CW // _NSPLIT

    def run(h_ref, src_ref, dst_ref):
        plsc.subcore_barrier()

        def gparts(j, k):
            return [
                pltpu.make_async_copy(
                    h_ref.at[sidx_v.at[j, pl.ds(p * part, part)]],
                    rows[k].at[pl.ds(p * part, part)], gsems[k])
                for p in range(_NSPLIT)
            ]

        def gstart(j, k):
            for g in gparts(j, k):
                g.start()

        def gwait(j, k):
            for g in gparts(j, k):
                g.wait()

        def seg(si, carry):
            base = r0 + si * _SLAB
            pltpu.sync_copy(src_ref.at[pl.ds(base, _SLAB)], sidx_v)
            pltpu.sync_copy(dst_ref.at[pl.ds(base, _SLAB)], didx_v)
            gstart(0, 0)

            def body2(t, carry2):
                j0 = 2 * t
                j1 = j0 + 1
                gstart(j1, 1)
                gwait(j0, 0)
                pltpu.sync_copy(rows0, acc.at[didx_v.at[j0]], add=True)

                @pl.when(j0 + 2 < _SLAB)
                def _():
                    gstart(j0 + 2, 0)
                gwait(j1, 1)
                pltpu.sync_copy(rows1, acc.at[didx_v.at[j1]], add=True)
                return carry2
            lax.fori_loop(0, _SLAB // 2, body2, 0)
            return carry
        lax.fori_loop(0, _RPW // _SLAB, seg, 0)

    @pl.when(c == 0)
    def _():
        run(hs1, src1, dst1)

    @pl.when(c == 1)
    def _():
        run(hs2, src2, dst2)

    plsc.subcore_barrier()

    @pl.when(c == 0)
    def _():
        _copy_rows(s, acc, o1)

    @pl.when(c == 1)
    def _():
        _copy_rows(s, acc, o2)


_NCHUNK = N // CH


def _sc_pool_body(h1t, h2t, batch1, batch2, zeros_nh, ones_hbm,
                  o_sum1, o_cnt1, o_sum2, o_cnt2,
                  acc_s, acc_c, rows_v, ones_v, idx_v):
    c = lax.axis_index("c")
    s = lax.axis_index("s")
    br = B // NS
    b0 = s * br
    pltpu.sync_copy(zeros_nh.at[pl.ds(b0, br)], acc_s.at[pl.ds(b0, br)])
    pltpu.sync_copy(zeros_nh.at[pl.ds(b0, br)], acc_c.at[pl.ds(b0, br)])
    pltpu.sync_copy(ones_hbm, ones_v)
    plsc.subcore_barrier()

    def run(h_ref, batch_ref):
        def body(t, carry):
            j = s + t * NS

            @pl.when(j < _NCHUNK)
            def _():
                b = j * CH
                pltpu.sync_copy(h_ref.at[pl.ds(b, CH)], rows_v)
                pltpu.sync_copy(batch_ref.at[pl.ds(b, CH)], idx_v)
                pltpu.sync_copy(rows_v, acc_s.at[idx_v], add=True)
                pltpu.sync_copy(ones_v, acc_c.at[idx_v], add=True)
            return carry
        lax.fori_loop(0, (_NCHUNK + NS - 1) // NS, body, 0)

    @pl.when(c == 0)
    def _():
        run(h1t, batch1)

    @pl.when(c == 1)
    def _():
        run(h2t, batch2)

    plsc.subcore_barrier()

    @pl.when(c == 0)
    def _():
        pltpu.sync_copy(acc_s.at[pl.ds(b0, br)], o_sum1.at[pl.ds(b0, br)])
        pltpu.sync_copy(acc_c.at[pl.ds(b0, br)], o_cnt1.at[pl.ds(b0, br)])

    @pl.when(c == 1)
    def _():
        pltpu.sync_copy(acc_s.at[pl.ds(b0, br)], o_sum2.at[pl.ds(b0, br)])
        pltpu.sync_copy(acc_c.at[pl.ds(b0, br)], o_cnt2.at[pl.ds(b0, br)])


@functools.cache
def _build_sc_deg():
    return pl.kernel(
        _sc_deg_body,
        out_type=[jax.ShapeDtypeStruct((N,), _f32)] * 4,
        mesh=_mesh(),
        scratch_types=[
            pltpu.VMEM_SHARED((N + 8,), _f32),
            pltpu.VMEM((_CW,), _f32),
            pltpu.VMEM((_NRB,), _f32),
            pltpu.VMEM((_NRB,), _f32),
            pltpu.VMEM((_SLAB, _CW), jnp.int32),
            pltpu.SemaphoreType.DMA,
            pltpu.SemaphoreType.DMA,
        ],
    )


@functools.cache
def _build_sc_agg():
    return pl.kernel(
        _sc_agg_body,
        out_type=[jax.ShapeDtypeStruct((N, H), _f32)] * 2,
        mesh=_mesh(),
        scratch_types=[
            pltpu.VMEM_SHARED((N + 8, H), _f32),
            pltpu.VMEM((_CW, H), _f32),
            pltpu.VMEM((_CW, H), _f32),
            pltpu.VMEM((_SLAB, _CW), jnp.int32),
            pltpu.VMEM((_SLAB, _CW), jnp.int32),
            pltpu.SemaphoreType.DMA,
            pltpu.SemaphoreType.DMA,
        ],
    )


@functools.cache
def _build_sc_pool():
    return pl.kernel(
        _sc_pool_body,
        out_type=[jax.ShapeDtypeStruct((B, H), _f32)] * 4,
        mesh=_mesh(),
        scratch_types=[
            pltpu.VMEM_SHARED((B, H), _f32),
            pltpu.VMEM_SHARED((B, H), _f32),
            pltpu.VMEM((CH, H), _f32),
            pltpu.VMEM((CH, H), _f32),
            pltpu.VMEM((CH,), jnp.int32),
        ],
    )


def _pad2d(idx, fill, width):
    pad = jnp.full((_EROWS * _CW - E,), fill, jnp.int32)
    return jnp.concatenate([idx.astype(jnp.int32), pad]).reshape(-1, width)


def _sc_deg(src1, dst1, src2, dst2):
    zeros_n = jnp.zeros((N,), _f32)
    ones_hbm = jnp.ones((_CW,), _f32)
    return _build_sc_deg()(_pad2d(src1, N, _CW), _pad2d(dst1, N, _CW),
                           _pad2d(src2, N, _CW), _pad2d(dst2, N, _CW),
                           zeros_n, ones_hbm)


def _sc_agg(hs1, hs2, src1, dst1, src2, dst2):
    zeros_nh = jnp.zeros((N, H), _f32)
    return _build_sc_agg()(hs1, hs2,
                           _pad2d(src1, 0, _CW), _pad2d(dst1, N, _CW),
                           _pad2d(src2, 0, _CW), _pad2d(dst2, N, _CW),
                           zeros_nh)


def _sc_pool(h1t, h2t, batch1, batch2):
    zeros_nh = jnp.zeros((N, H), _f32)
    ones_hbm = jnp.ones((CH, H), _f32)
    return _build_sc_pool()(h1t, h2t, batch1, batch2, zeros_nh, ones_hbm)


_ROWS = 1000


def _tc_prep(h1, hist_s1, h2, hist_s2):
    def body(h1_ref, hh1_ref, h2_ref, hh2_ref, o1_ref, o2_ref):
        for h_ref, hh_ref, o_ref in ((h1_ref, hh1_ref, o1_ref),
                                     (h2_ref, hh2_ref, o2_ref)):
            deg = jnp.maximum(hh_ref[...], 1.0)
            o_ref[...] = h_ref[...] * lax.rsqrt(deg)

    grid = (N // _ROWS,)
    row_spec = pl.BlockSpec((_ROWS, H), lambda i: (i, 0))
    col_spec = pl.BlockSpec((_ROWS, 1), lambda i: (i, 0))
    return pl.pallas_call(
        body,
        grid=grid,
        in_specs=[row_spec, col_spec, row_spec, col_spec],
        out_specs=[row_spec, row_spec],
        out_shape=[jax.ShapeDtypeStruct((N, H), _f32)] * 2,
    )(h1, hist_s1, h2, hist_s2)


def _tc_conv_dense(agg1, hist_d1, hist_s1, agg2, hist_d2, hist_s2, W, b,
                   scale_out):
    def body(a1_ref, hd1_ref, hs1_ref, a2_ref, hd2_ref, hs2_ref, w_ref, b_ref,
             o1_ref, o2_ref):
        w = w_ref[...]
        bias = b_ref[...]
        for a_ref, hd_ref, hs_ref, o_ref in (
                (a1_ref, hd1_ref, hs1_ref, o1_ref),
                (a2_ref, hd2_ref, hs2_ref, o2_ref)):
            sin = lax.rsqrt(jnp.maximum(hd_ref[...], 1.0))
            h = jnp.dot(a_ref[...] * sin, w, preferred_element_type=_f32, precision=lax.Precision.HIGHEST)
            h = jnp.maximum(h + bias, 0.0)
            if scale_out:
                h = h * lax.rsqrt(jnp.maximum(hs_ref[...], 1.0))
            o_ref[...] = h

    grid = (N // _ROWS,)
    row_spec = pl.BlockSpec((_ROWS, H), lambda i: (i, 0))
    col_spec = pl.BlockSpec((_ROWS, 1), lambda i: (i, 0))
    w_spec = pl.BlockSpec((H, H), lambda i: (0, 0))
    b_spec = pl.BlockSpec((1, H), lambda i: (0, 0))
    return pl.pallas_call(
        body,
        grid=grid,
        in_specs=[row_spec, col_spec, col_spec,
                  row_spec, col_spec, col_spec, w_spec, b_spec],
        out_specs=[row_spec, row_spec],
        out_shape=[jax.ShapeDtypeStruct((N, H), _f32)] * 2,
    )(agg1, hist_d1, hist_s1, agg2, hist_d2, hist_s2, W, b)


_BR = 128


def _tc_tail(sum1, cnt1, sum2, cnt2, scal, wp):
    def body(s1_ref, c1_ref, s2_ref, c2_ref, sc_ref,
             projW_ref, projb_ref, enw1_ref, enb1_ref, wt_ref, b2r_ref,
             nnb_ref, wihr_ref, wihz_ref, wihn_ref, bihr_ref, bihz_ref,
             bihn_ref, whhr_ref, whhz_ref, whhn_ref, bhhr_ref, bhhz_ref,
             bhhn_ref, c1a_ref, c1b_ref, c1bias_ref, c2w_ref, c2b_ref,
             c3w_ref, c3b_ref, o_ref):
        scalv = sc_ref[...]
        solv = scalv[:, 0:1]
        inter = scalv[:, 1:2]
        intra1 = scalv[:, 2:3]
        intra2 = scalv[:, 3:4]

        hg1 = solv * (s1_ref[...] / jnp.maximum(c1_ref[...][:, 0:1], 1.0))
        hg2 = (1.0 - solv) * (s2_ref[...] / jnp.maximum(c2_ref[...][:, 0:1], 1.0))

        projW = projW_ref[...]
        projb = projb_ref[...]
        x1 = jnp.maximum(jnp.dot(hg1, projW, preferred_element_type=_f32, precision=lax.Precision.HIGHEST) + projb, 0.0)
        x2 = jnp.maximum(jnp.dot(hg2, projW, preferred_element_type=_f32, precision=lax.Precision.HIGHEST) + projb, 0.0)

        wt = wt_ref[...]
        t1 = jnp.dot(x1, wt, preferred_element_type=_f32, precision=lax.Precision.HIGHEST)
        t2 = jnp.dot(x2, wt, preferred_element_type=_f32, precision=lax.Precision.HIGHEST)

        enw1 = enw1_ref[...]
        enb1 = enb1_ref[...]
        a_inter = jnp.maximum(inter * enw1 + enb1, 0.0)
        a_i1 = jnp.maximum(intra1 * enw1 + enb1, 0.0)
        a_i2 = jnp.maximum(intra2 * enw1 + enb1, 0.0)

        def contract(amat, t):
            acc = amat[:, 0:1] * t[:, 0:H]
            for k in range(1, 32):
                acc = acc + amat[:, k:k + 1] * t[:, k * H:(k + 1) * H]
            return acc

        b2r = b2r_ref[...]
        xb1 = jnp.dot(x1, b2r, preferred_element_type=_f32, precision=lax.Precision.HIGHEST)
        xb2 = jnp.dot(x2, b2r, preferred_element_type=_f32, precision=lax.Precision.HIGHEST)
        nnb = nnb_ref[...]
        agg1 = contract(a_inter, t2) + xb2 + contract(a_i1, t1) + xb1 + nnb
        agg2 = contract(a_inter, t1) + xb1 + contract(a_i2, t2) + xb2 + nnb
        xt1 = jnp.maximum(agg1, 0.0)
        xt2 = jnp.maximum(agg2, 0.0)

        wihr, wihz, wihn = wihr_ref[...], wihz_ref[...], wihn_ref[...]
        whhr, whhz, whhn = whhr_ref[...], whhz_ref[...], whhn_ref[...]
        bihr, bihz, bihn = bihr_ref[...], bihz_ref[...], bihn_ref[...]
        bhhr, bhhz, bhhn = bhhr_ref[...], bhhz_ref[...], bhhn_ref[...]

        def gru(xt, hid):
            ir = jnp.dot(xt, wihr, preferred_element_type=_f32, precision=lax.Precision.HIGHEST) + bihr
            iz = jnp.dot(xt, wihz, preferred_element_type=_f32, precision=lax.Precision.HIGHEST) + bihz
            inn = jnp.dot(xt, wihn, preferred_element_type=_f32, precision=lax.Precision.HIGHEST) + bihn
            hr = jnp.dot(hid, whhr, preferred_element_type=_f32, precision=lax.Precision.HIGHEST) + bhhr
            hz = jnp.dot(hid, whhz, preferred_element_type=_f32, precision=lax.Precision.HIGHEST) + bhhz
            hn = jnp.dot(hid, whhn, preferred_element_type=_f32, precision=lax.Precision.HIGHEST) + bhhn
            r = jax.nn.sigmoid(ir + hr)
            z = jax.nn.sigmoid(iz + hz)
            nn_ = jnp.tanh(inn + r * hn)
            return (1.0 - z) * nn_ + z * hid

        hid1 = gru(xt1, x1)
        hid2 = gru(xt2, x2)

        o = jnp.dot(hid1, c1a_ref[...], preferred_element_type=_f32, precision=lax.Precision.HIGHEST)
        o = o + jnp.dot(hid2, c1b_ref[...], preferred_element_type=_f32, precision=lax.Precision.HIGHEST)
        o = jnp.maximum(o + c1bias_ref[...], 0.0)
        o = jnp.maximum(
            jnp.dot(o, c2w_ref[...], preferred_element_type=_f32, precision=lax.Precision.HIGHEST) + c2b_ref[...], 0.0)
        o_ref[...] = jnp.dot(o, c3w_ref[...], preferred_element_type=_f32, precision=lax.Precision.HIGHEST) + c3b_ref[...]

    grid = (B // _BR,)
    row_spec = pl.BlockSpec((_BR, H), lambda i: (i, 0))
    cnt_spec = row_spec
    scal_spec = pl.BlockSpec((_BR, 8), lambda i: (i, 0))

    def const2(shape):
        return pl.BlockSpec(shape, lambda i: (0, 0))

    in_specs = [row_spec, cnt_spec, row_spec, cnt_spec, scal_spec,
                const2((H, H)), const2((1, H)), const2((1, 32)),
                const2((1, 32)), const2((H, 32 * H)), const2((H, H)),
                const2((1, H))]
    in_specs += [const2((H, H))] * 3 + [const2((1, H))] * 3
    in_specs += [const2((H, H))] * 3 + [const2((1, H))] * 3
    in_specs += [const2((H, H)), const2((H, H)), const2((1, H)),
                 const2((H, H)), const2((1, H)), const2((H, H)),
                 const2((1, H))]
    return pl.pallas_call(
        body,
        grid=grid,
        in_specs=in_specs,
        out_specs=pl.BlockSpec((_BR, H), lambda i: (i, 0)),
        out_shape=jax.ShapeDtypeStruct((B, H), _f32),
    )(sum1, cnt1, sum2, cnt2, scal, *wp)


def kernel(h1, h2, solv1_x, inter_hb, intra_hb1, intra_hb2, params,
           edge_index1, edge_index2, batch1, batch2):
    src1, dst1 = edge_index1[0], edge_index1[1]
    src2, dst2 = edge_index2[0], edge_index2[1]
    ssrc1, sdst1 = lax.sort_key_val(src1, dst1)
    ssrc2, sdst2 = lax.sort_key_val(src2, dst2)

    hs1_hist, hd1_hist, hs2_hist, hd2_hist = (
        h[:, None] for h in _sc_deg(src1, dst1, src2, dst2))

    hs1, hs2 = _tc_prep(h1, hs1_hist, h2, hs2_hist)

    agg1, agg2 = _sc_agg(hs1, hs2, ssrc1, sdst1, ssrc2, sdst2)
    hsl2_1, hsl2_2 = _tc_conv_dense(
        agg1, hd1_hist, hs1_hist, agg2, hd2_hist, hs2_hist,
        params['conv1_W'], params['conv1_b'][None, :], scale_out=True)

    agg1b, agg2b = _sc_agg(hsl2_1, hsl2_2, ssrc1, sdst1, ssrc2, sdst2)
    h1t, h2t = _tc_conv_dense(
        agg1b, hd1_hist, hs1_hist, agg2b, hd2_hist, hs2_hist,
        params['conv2_W'], params['conv2_b'][None, :], scale_out=False)

    sum1, cnt1, sum2, cnt2 = _sc_pool(h1t, h2t, batch1, batch2)

    scal = jnp.zeros((B, 8), _f32)
    scal = scal.at[:, 0].set(solv1_x)
    scal = scal.at[:, 1].set(inter_hb)
    scal = scal.at[:, 2].set(intra_hb1)
    scal = scal.at[:, 3].set(intra_hb2)

    p = params
    wt = p['en_W2'].reshape(32, H, H).transpose(1, 0, 2).reshape(H, 32 * H)
    b2r = p['en_b2'].reshape(H, H)
    gih = p['gru_Wih']
    ghh = p['gru_Whh']
    wp = (
        p['proj_W'], p['proj_b'][None, :], p['en_W1'], p['en_b1'][None, :],
        wt, b2r, p['nn_bias'][None, :],
        gih[0:H].T, gih[H:2 * H].T, gih[2 * H:3 * H].T,
        p['gru_bih'][None, 0:H], p['gru_bih'][None, H:2 * H],
        p['gru_bih'][None, 2 * H:3 * H],
        ghh[0:H].T, ghh[H:2 * H].T, ghh[2 * H:3 * H].T,
        p['gru_bhh'][None, 0:H], p['gru_bhh'][None, H:2 * H],
        p['gru_bhh'][None, 2 * H:3 * H],
        p['c1_W'][0:H], p['c1_W'][H:2 * H], p['c1_b'][None, :],
        p['c2_W'], p['c2_b'][None, :],
        jnp.pad(p['c3_W'], ((0, 0), (0, H - p['c3_W'].shape[1]))),
        jnp.pad(p['c3_b'], (0, H - p['c3_b'].shape[0]))[None, :],
    )
    outp = _tc_tail(sum1, cnt1, sum2, cnt2, scal, wp)
    return outp[:, :2]


# speedup vs baseline: 1.6401x; 1.6401x over previous
import functools

import jax
import jax.numpy as jnp
from jax import lax
from jax.experimental import pallas as pl
from jax.experimental.pallas import tpu as pltpu
from jax.experimental.pallas import tpu_sc as plsc

N = 10000
E = 320000
B = 1024
H = 128
NC = 2
NS = 16
CH = 80

_f32 = jnp.float32


def _mesh():
    return plsc.VectorSubcoreMesh(
        core_axis_name="c", subcore_axis_name="s", num_cores=NC, num_subcores=NS
    )


_NRB = (N // NS) // 8 * 8
_NTAIL = N - _NRB * NS


def _copy_rows(s, src_ref, dst_ref):
    pltpu.sync_copy(src_ref.at[pl.ds(s * _NRB, _NRB)],
                    dst_ref.at[pl.ds(s * _NRB, _NRB)])

    @pl.when(s == NS - 1)
    def _():
        pltpu.sync_copy(src_ref.at[pl.ds(_NRB * NS, _NTAIL)],
                        dst_ref.at[pl.ds(_NRB * NS, _NTAIL)])


_copy_elems = _copy_rows


_EROWS = 2560
_RPW = _EROWS // NS
_CW = 128
_SLAB = 32


def _sc_deg_body(src1, dst1, src2, dst2, zeros_n, ones_hbm,
                 o_s1, o_d1, o_s2, o_d2, hist, ones_v, zbuf, obuf, idx_v,
                 ssem0, ssem1):
    c = lax.axis_index("c")
    s = lax.axis_index("s")
    pltpu.sync_copy(ones_hbm, ones_v)
    pltpu.sync_copy(zeros_n.at[pl.ds(0, _NRB)], zbuf)
    r0 = s * _RPW
    e0 = s * _NRB

    def phase(idx_ref, out_ref):
        pltpu.sync_copy(zbuf, hist.at[pl.ds(e0, _NRB)])

        @pl.when(s == NS - 1)
        def _():
            pltpu.sync_copy(zbuf.at[pl.ds(0, _NTAIL)],
                            hist.at[pl.ds(_NRB * NS, _NTAIL)])
        plsc.subcore_barrier()

        def start(j, sem):
            pltpu.async_copy(ones_v, hist.at[idx_v.at[j]], sem, add=True)

        def wait(j, sem):
            pltpu.make_async_copy(ones_v, hist.at[idx_v.at[j]], sem).wait()

        def seg(si, carry):
            pltpu.sync_copy(idx_ref.at[pl.ds(r0 + si * _SLAB, _SLAB)], idx_v)
            start(0, ssem0)

            def body2(t, carry2):
                j0 = 2 * t
                j1 = j0 + 1
                start(j1, ssem1)
                wait(j0, ssem0)

                @pl.when(j0 + 2 < _SLAB)
                def _():
                    start(j0 + 2, ssem0)
                wait(j1, ssem1)
                return carry2
            lax.fori_loop(0, _SLAB // 2, body2, 0)
            return carry
        lax.fori_loop(0, _RPW // _SLAB, seg, 0)
        plsc.subcore_barrier()
        pltpu.sync_copy(hist.at[pl.ds(e0, _NRB)], obuf)
        pltpu.sync_copy(obuf, out_ref.at[pl.ds(e0, _NRB)])

        @pl.when(s == NS - 1)
        def _():
            pltpu.sync_copy(hist.at[pl.ds(_NRB * NS, _NTAIL)],
                            obuf.at[pl.ds(0, _NTAIL)])
            pltpu.sync_copy(obuf.at[pl.ds(0, _NTAIL)],
                            out_ref.at[pl.ds(_NRB * NS, _NTAIL)])

    @pl.when(c == 0)
    def _():
        phase(src1, o_s1)
        phase(dst1, o_d1)

    @pl.when(c == 1)
    def _():
        phase(src2, o_s2)
        phase(dst2, o_d2)


_NSPLIT = 2


def _sc_agg_body(hs1, hs2, src1, dst1, src2, dst2, zeros_nh,
                 o1, o2, acc, rows0, rows1, sidx_v, didx_v, gsem0, gsem1):
    c = lax.axis_index("c")
    s = lax.axis_index("s")
    _copy_rows(s, zeros_nh, acc)
    r0 = s * _RPW
    rows = (rows0, rows1)
    gsems = (gsem0, gsem1)
    part = _CW // _NSPLIT

    def run(h_ref, src_ref, dst_ref):
        plsc.subcore_barrier()

        def gparts(j, k):
            return [
                pltpu.make_async_copy(
                    h_ref.at[sidx_v.at[j, pl.ds(p * part, part)]],
                    rows[k].at[pl.ds(p * part, part)], gsems[k])
                for p in range(_NSPLIT)
            ]

        def gstart(j, k):
            for g in gparts(j, k):
                g.start()

        def gwait(j, k):
            for g in gparts(j, k):
                g.wait()

        def seg(si, carry):
            base = r0 + si * _SLAB
            pltpu.sync_copy(src_ref.at[pl.ds(base, _SLAB)], sidx_v)
            pltpu.sync_copy(dst_ref.at[pl.ds(base, _SLAB)], didx_v)
            gstart(0, 0)

            def body2(t, carry2):
                j0 = 2 * t
                j1 = j0 + 1
                gstart(j1, 1)
                gwait(j0, 0)
                pltpu.sync_copy(rows0, acc.at[didx_v.at[j0]], add=True)

                @pl.when(j0 + 2 < _SLAB)
                def _():
                    gstart(j0 + 2, 0)
                gwait(j1, 1)
                pltpu.sync_copy(rows1, acc.at[didx_v.at[j1]], add=True)
                return carry2
            lax.fori_loop(0, _SLAB // 2, body2, 0)
            return carry
        lax.fori_loop(0, _RPW // _SLAB, seg, 0)

    @pl.when(c == 0)
    def _():
        run(hs1, src1, dst1)

    @pl.when(c == 1)
    def _():
        run(hs2, src2, dst2)

    plsc.subcore_barrier()

    @pl.when(c == 0)
    def _():
        _copy_rows(s, acc, o1)

    @pl.when(c == 1)
    def _():
        _copy_rows(s, acc, o2)


_NCHUNK = N // CH


def _sc_pool_body(h1t, h2t, batch1, batch2, zeros_nh, ones_hbm,
                  o_sum1, o_cnt1, o_sum2, o_cnt2,
                  acc_s, acc_c, rows_v, ones_v, idx_v):
    c = lax.axis_index("c")
    s = lax.axis_index("s")
    br = B // NS
    b0 = s * br
    pltpu.sync_copy(zeros_nh.at[pl.ds(b0, br)], acc_s.at[pl.ds(b0, br)])
    pltpu.sync_copy(zeros_nh.at[pl.ds(b0, br)], acc_c.at[pl.ds(b0, br)])
    pltpu.sync_copy(ones_hbm, ones_v)
    plsc.subcore_barrier()

    def run(h_ref, batch_ref):
        def body(t, carry):
            j = s + t * NS

            @pl.when(j < _NCHUNK)
            def _():
                b = j * CH
                pltpu.sync_copy(h_ref.at[pl.ds(b, CH)], rows_v)
                pltpu.sync_copy(batch_ref.at[pl.ds(b, CH)], idx_v)
                pltpu.sync_copy(rows_v, acc_s.at[idx_v], add=True)
                pltpu.sync_copy(ones_v, acc_c.at[idx_v], add=True)
            return carry
        lax.fori_loop(0, (_NCHUNK + NS - 1) // NS, body, 0)

    @pl.when(c == 0)
    def _():
        run(h1t, batch1)

    @pl.when(c == 1)
    def _():
        run(h2t, batch2)

    plsc.subcore_barrier()

    @pl.when(c == 0)
    def _():
        pltpu.sync_copy(acc_s.at[pl.ds(b0, br)], o_sum1.at[pl.ds(b0, br)])
        pltpu.sync_copy(acc_c.at[pl.ds(b0, br)], o_cnt1.at[pl.ds(b0, br)])

    @pl.when(c == 1)
    def _():
        pltpu.sync_copy(acc_s.at[pl.ds(b0, br)], o_sum2.at[pl.ds(b0, br)])
        pltpu.sync_copy(acc_c.at[pl.ds(b0, br)], o_cnt2.at[pl.ds(b0, br)])


@functools.cache
def _build_sc_deg():
    return pl.kernel(
        _sc_deg_body,
        out_type=[jax.ShapeDtypeStruct((N,), _f32)] * 4,
        mesh=_mesh(),
        scratch_types=[
            pltpu.VMEM_SHARED((N + 8,), _f32),
            pltpu.VMEM((_CW,), _f32),
            pltpu.VMEM((_NRB,), _f32),
            pltpu.VMEM((_NRB,), _f32),
            pltpu.VMEM((_SLAB, _CW), jnp.int32),
            pltpu.SemaphoreType.DMA,
            pltpu.SemaphoreType.DMA,
        ],
    )


@functools.cache
def _build_sc_agg():
    return pl.kernel(
        _sc_agg_body,
        out_type=[jax.ShapeDtypeStruct((N, H), _f32)] * 2,
        mesh=_mesh(),
        scratch_types=[
            pltpu.VMEM_SHARED((N + 8, H), _f32),
            pltpu.VMEM((_CW, H), _f32),
            pltpu.VMEM((_CW, H), _f32),
            pltpu.VMEM((_SLAB, _CW), jnp.int32),
            pltpu.VMEM((_SLAB, _CW), jnp.int32),
            pltpu.SemaphoreType.DMA,
            pltpu.SemaphoreType.DMA,
        ],
    )


@functools.cache
def _build_sc_pool():
    return pl.kernel(
        _sc_pool_body,
        out_type=[jax.ShapeDtypeStruct((B, H), _f32)] * 4,
        mesh=_mesh(),
        scratch_types=[
            pltpu.VMEM_SHARED((B, H), _f32),
            pltpu.VMEM_SHARED((B, H), _f32),
            pltpu.VMEM((CH, H), _f32),
            pltpu.VMEM((CH, H), _f32),
            pltpu.VMEM((CH,), jnp.int32),
        ],
    )


def _pad2d(idx, fill, width):
    pad = jnp.full((_EROWS * _CW - E,), fill, jnp.int32)
    return jnp.concatenate([idx.astype(jnp.int32), pad]).reshape(-1, width)


def _sc_deg(src1, dst1, src2, dst2):
    zeros_n = jnp.zeros((N,), _f32)
    ones_hbm = jnp.ones((_CW,), _f32)
    return _build_sc_deg()(_pad2d(src1, N, _CW), _pad2d(dst1, N, _CW),
                           _pad2d(src2, N, _CW), _pad2d(dst2, N, _CW),
                           zeros_n, ones_hbm)


def _sc_agg(hs1, hs2, src1, dst1, src2, dst2):
    zeros_nh = jnp.zeros((N, H), _f32)
    return _build_sc_agg()(hs1, hs2,
                           _pad2d(src1, 0, _CW), _pad2d(dst1, N, _CW),
                           _pad2d(src2, 0, _CW), _pad2d(dst2, N, _CW),
                           zeros_nh)


def _sc_pool(h1t, h2t, batch1, batch2):
    zeros_nh = jnp.zeros((N, H), _f32)
    ones_hbm = jnp.ones((CH, H), _f32)
    return _build_sc_pool()(h1t, h2t, batch1, batch2, zeros_nh, ones_hbm)


_ROWS = 1000


def _tc_prep(h1, hist_s1, h2, hist_s2):
    def body(h1_ref, hh1_ref, h2_ref, hh2_ref, o1_ref, o2_ref):
        for h_ref, hh_ref, o_ref in ((h1_ref, hh1_ref, o1_ref),
                                     (h2_ref, hh2_ref, o2_ref)):
            deg = jnp.maximum(hh_ref[...], 1.0)
            o_ref[...] = h_ref[...] * lax.rsqrt(deg)

    grid = (N // _ROWS,)
    row_spec = pl.BlockSpec((_ROWS, H), lambda i: (i, 0))
    col_spec = pl.BlockSpec((_ROWS, 1), lambda i: (i, 0))
    return pl.pallas_call(
        body,
        grid=grid,
        in_specs=[row_spec, col_spec, row_spec, col_spec],
        out_specs=[row_spec, row_spec],
        out_shape=[jax.ShapeDtypeStruct((N, H), _f32)] * 2,
    )(h1, hist_s1, h2, hist_s2)


def _tc_conv_dense(agg1, hist_d1, hist_s1, agg2, hist_d2, hist_s2, W, b,
                   scale_out):
    def body(a1_ref, hd1_ref, hs1_ref, a2_ref, hd2_ref, hs2_ref, w_ref, b_ref,
             o1_ref, o2_ref):
        w = w_ref[...]
        bias = b_ref[...]
        for a_ref, hd_ref, hs_ref, o_ref in (
                (a1_ref, hd1_ref, hs1_ref, o1_ref),
                (a2_ref, hd2_ref, hs2_ref, o2_ref)):
            sin = lax.rsqrt(jnp.maximum(hd_ref[...], 1.0))
            h = jnp.dot(a_ref[...] * sin, w, preferred_element_type=_f32)
            h = jnp.maximum(h + bias, 0.0)
            if scale_out:
                h = h * lax.rsqrt(jnp.maximum(hs_ref[...], 1.0))
            o_ref[...] = h

    grid = (N // _ROWS,)
    row_spec = pl.BlockSpec((_ROWS, H), lambda i: (i, 0))
    col_spec = pl.BlockSpec((_ROWS, 1), lambda i: (i, 0))
    w_spec = pl.BlockSpec((H, H), lambda i: (0, 0))
    b_spec = pl.BlockSpec((1, H), lambda i: (0, 0))
    return pl.pallas_call(
        body,
        grid=grid,
        in_specs=[row_spec, col_spec, col_spec,
                  row_spec, col_spec, col_spec, w_spec, b_spec],
        out_specs=[row_spec, row_spec],
        out_shape=[jax.ShapeDtypeStruct((N, H), _f32)] * 2,
    )(agg1, hist_d1, hist_s1, agg2, hist_d2, hist_s2, W, b)


_BR = 128


def _tc_tail(sum1, cnt1, sum2, cnt2, scal, wp):
    def body(s1_ref, c1_ref, s2_ref, c2_ref, sc_ref,
             projW_ref, projb_ref, enw1_ref, enb1_ref, wt_ref, b2r_ref,
             nnb_ref, wihr_ref, wihz_ref, wihn_ref, bihr_ref, bihz_ref,
             bihn_ref, whhr_ref, whhz_ref, whhn_ref, bhhr_ref, bhhz_ref,
             bhhn_ref, c1a_ref, c1b_ref, c1bias_ref, c2w_ref, c2b_ref,
             c3w_ref, c3b_ref, o_ref):
        scalv = sc_ref[...]
        solv = scalv[:, 0:1]
        inter = scalv[:, 1:2]
        intra1 = scalv[:, 2:3]
        intra2 = scalv[:, 3:4]

        hg1 = solv * (s1_ref[...] / jnp.maximum(c1_ref[...][:, 0:1], 1.0))
        hg2 = (1.0 - solv) * (s2_ref[...] / jnp.maximum(c2_ref[...][:, 0:1], 1.0))

        projW = projW_ref[...]
        projb = projb_ref[...]
        x1 = jnp.maximum(jnp.dot(hg1, projW, preferred_element_type=_f32) + projb, 0.0)
        x2 = jnp.maximum(jnp.dot(hg2, projW, preferred_element_type=_f32) + projb, 0.0)

        wt = wt_ref[...]
        t1 = jnp.dot(x1, wt, preferred_element_type=_f32, precision=lax.Precision.HIGHEST)
        t2 = jnp.dot(x2, wt, preferred_element_type=_f32, precision=lax.Precision.HIGHEST)

        enw1 = enw1_ref[...]
        enb1 = enb1_ref[...]
        a_inter = jnp.maximum(inter * enw1 + enb1, 0.0)
        a_i1 = jnp.maximum(intra1 * enw1 + enb1, 0.0)
        a_i2 = jnp.maximum(intra2 * enw1 + enb1, 0.0)

        def contract(amat, t):
            acc = amat[:, 0:1] * t[:, 0:H]
            for k in range(1, 32):
                acc = acc + amat[:, k:k + 1] * t[:, k * H:(k + 1) * H]
            return acc

        b2r = b2r_ref[...]
        xb1 = jnp.dot(x1, b2r, preferred_element_type=_f32, precision=lax.Precision.HIGHEST)
        xb2 = jnp.dot(x2, b2r, preferred_element_type=_f32, precision=lax.Precision.HIGHEST)
        nnb = nnb_ref[...]
        agg1 = contract(a_inter, t2) + xb2 + contract(a_i1, t1) + xb1 + nnb
        agg2 = contract(a_inter, t1) + xb1 + contract(a_i2, t2) + xb2 + nnb
        xt1 = jnp.maximum(agg1, 0.0)
        xt2 = jnp.maximum(agg2, 0.0)

        wihr, wihz, wihn = wihr_ref[...], wihz_ref[...], wihn_ref[...]
        whhr, whhz, whhn = whhr_ref[...], whhz_ref[...], whhn_ref[...]
        bihr, bihz, bihn = bihr_ref[...], bihz_ref[...], bihn_ref[...]
        bhhr, bhhz, bhhn = bhhr_ref[...], bhhz_ref[...], bhhn_ref[...]

        def gru(xt, hid):
            ir = jnp.dot(xt, wihr, preferred_element_type=_f32) + bihr
            iz = jnp.dot(xt, wihz, preferred_element_type=_f32) + bihz
            inn = jnp.dot(xt, wihn, preferred_element_type=_f32) + bihn
            hr = jnp.dot(hid, whhr, preferred_element_type=_f32) + bhhr
            hz = jnp.dot(hid, whhz, preferred_element_type=_f32) + bhhz
            hn = jnp.dot(hid, whhn, preferred_element_type=_f32) + bhhn
            r = jax.nn.sigmoid(ir + hr)
            z = jax.nn.sigmoid(iz + hz)
            nn_ = jnp.tanh(inn + r * hn)
            return (1.0 - z) * nn_ + z * hid

        hid1 = gru(xt1, x1)
        hid2 = gru(xt2, x2)

        o = jnp.dot(hid1, c1a_ref[...], preferred_element_type=_f32)
        o = o + jnp.dot(hid2, c1b_ref[...], preferred_element_type=_f32)
        o = jnp.maximum(o + c1bias_ref[...], 0.0)
        o = jnp.maximum(
            jnp.dot(o, c2w_ref[...], preferred_element_type=_f32) + c2b_ref[...], 0.0)
        o_ref[...] = jnp.dot(o, c3w_ref[...], preferred_element_type=_f32) + c3b_ref[...]

    grid = (B // _BR,)
    row_spec = pl.BlockSpec((_BR, H), lambda i: (i, 0))
    cnt_spec = row_spec
    scal_spec = pl.BlockSpec((_BR, 8), lambda i: (i, 0))

    def const2(shape):
        return pl.BlockSpec(shape, lambda i: (0, 0))

    in_specs = [row_spec, cnt_spec, row_spec, cnt_spec, scal_spec,
                const2((H, H)), const2((1, H)), const2((1, 32)),
                const2((1, 32)), const2((H, 32 * H)), const2((H, H)),
                const2((1, H))]
    in_specs += [const2((H, H))] * 3 + [const2((1, H))] * 3
    in_specs += [const2((H, H))] * 3 + [const2((1, H))] * 3
    in_specs += [const2((H, H)), const2((H, H)), const2((1, H)),
                 const2((H, H)), const2((1, H)), const2((H, H)),
                 const2((1, H))]
    return pl.pallas_call(
        body,
        grid=grid,
        in_specs=in_specs,
        out_specs=pl.BlockSpec((_BR, H), lambda i: (i, 0)),
        out_shape=jax.ShapeDtypeStruct((B, H), _f32),
    )(sum1, cnt1, sum2, cnt2, scal, *wp)


def kernel(h1, h2, solv1_x, inter_hb, intra_hb1, intra_hb2, params,
           edge_index1, edge_index2, batch1, batch2):
    src1, dst1 = edge_index1[0], edge_index1[1]
    src2, dst2 = edge_index2[0], edge_index2[1]

    hs1_hist, hd1_hist, hs2_hist, hd2_hist = (
        h[:, None] for h in _sc_deg(src1, dst1, src2, dst2))

    hs1, hs2 = _tc_prep(h1, hs1_hist, h2, hs2_hist)

    agg1, agg2 = _sc_agg(hs1, hs2, src1, dst1, src2, dst2)
    hsl2_1, hsl2_2 = _tc_conv_dense(
        agg1, hd1_hist, hs1_hist, agg2, hd2_hist, hs2_hist,
        params['conv1_W'], params['conv1_b'][None, :], scale_out=True)

    agg1b, agg2b = _sc_agg(hsl2_1, hsl2_2, src1, dst1, src2, dst2)
    h1t, h2t = _tc_conv_dense(
        agg1b, hd1_hist, hs1_hist, agg2b, hd2_hist, hs2_hist,
        params['conv2_W'], params['conv2_b'][None, :], scale_out=False)

    sum1, cnt1, sum2, cnt2 = _sc_pool(h1t, h2t, batch1, batch2)

    scal = jnp.zeros((B, 8), _f32)
    scal = scal.at[:, 0].set(solv1_x)
    scal = scal.at[:, 1].set(inter_hb)
    scal = scal.at[:, 2].set(intra_hb1)
    scal = scal.at[:, 3].set(intra_hb2)

    p = params
    wt = p['en_W2'].reshape(32, H, H).transpose(1, 0, 2).reshape(H, 32 * H)
    b2r = p['en_b2'].reshape(H, H)
    gih = p['gru_Wih']
    ghh = p['gru_Whh']
    wp = (
        p['proj_W'], p['proj_b'][None, :], p['en_W1'], p['en_b1'][None, :],
        wt, b2r, p['nn_bias'][None, :],
        gih[0:H].T, gih[H:2 * H].T, gih[2 * H:3 * H].T,
        p['gru_bih'][None, 0:H], p['gru_bih'][None, H:2 * H],
        p['gru_bih'][None, 2 * H:3 * H],
        ghh[0:H].T, ghh[H:2 * H].T, ghh[2 * H:3 * H].T,
        p['gru_bhh'][None, 0:H], p['gru_bhh'][None, H:2 * H],
        p['gru_bhh'][None, 2 * H:3 * H],
        p['c1_W'][0:H], p['c1_W'][H:2 * H], p['c1_b'][None, :],
        p['c2_W'], p['c2_b'][None, :],
        jnp.pad(p['c3_W'], ((0, 0), (0, H - p['c3_W'].shape[1]))),
        jnp.pad(p['c3_b'], (0, H - p['c3_b'].shape[0]))[None, :],
    )
    outp = _tc_tail(sum1, cnt1, sum2, cnt2, scal, wp)
    return outp[:, :2]
